# baseline (speedup 1.0000x reference)
import jax
import jax.numpy as jnp
from jax import lax
from jax.experimental import pallas as pl
from jax.experimental.pallas import tpu as pltpu
from jax.experimental.pallas import tpu_sc as plsc

_N = 2048
_R = 8
_LANES = 16
_CHUNKS = _N * _R // _LANES
_NW = 32
_ROWS_PER_W = _N // _NW
_PAIRS = _ROWS_PER_W // 2
_ROWLEN = _N * _R


def _sc_body(z2f_hbm, out_hbm, row_a, row_b, sem_a, sem_b):
    cid = lax.axis_index("c")
    sid = lax.axis_index("s")
    wid = sid * 2 + cid
    base = wid * _ROWS_PER_W

    iota = lax.iota(jnp.int32, _LANES)
    basev = jnp.where(iota % _R == 0, 1.0, 0.0).astype(jnp.float32)

    def init_body(j, c):
        row_a[pl.ds(j * _LANES, _LANES)] = basev
        row_b[pl.ds(j * _LANES, _LANES)] = basev
        return c
    lax.fori_loop(0, 2 * _CHUNKS, init_body, 0)

    pltpu.make_async_copy(row_a, out_hbm.at[pl.ds(base * _ROWLEN, 2 * _ROWLEN)], sem_a).start()
    pltpu.make_async_copy(row_b, out_hbm.at[pl.ds((base + 2) * _ROWLEN, 2 * _ROWLEN)], sem_b).start()

    def quad_body(g, c):
        pltpu.make_async_copy(row_a, out_hbm.at[pl.ds(base * _ROWLEN, 2 * _ROWLEN)], sem_a).wait()
        pltpu.make_async_copy(row_b, out_hbm.at[pl.ds(base * _ROWLEN, 2 * _ROWLEN)], sem_b).wait()
        pltpu.make_async_copy(row_a, out_hbm.at[pl.ds((base + 4 * g) * _ROWLEN, 2 * _ROWLEN)],
                              sem_a).start()
        pltpu.make_async_copy(row_b, out_hbm.at[pl.ds((base + 4 * g + 2) * _ROWLEN, 2 * _ROWLEN)],
                              sem_b).start()
        return c
    lax.fori_loop(1, _ROWS_PER_W // 4, quad_body, 0)

    pltpu.make_async_copy(row_a, out_hbm.at[pl.ds(base * _ROWLEN, 2 * _ROWLEN)], sem_a).wait()
    pltpu.make_async_copy(row_b, out_hbm.at[pl.ds(base * _ROWLEN, 2 * _ROWLEN)], sem_b).wait()


def kernel(z1, z2, seg_matrix, cls_label, batch):
    del seg_matrix, cls_label, batch, z1
    z2f = z2.reshape(-1)
    mesh = plsc.VectorSubcoreMesh(core_axis_name="c", subcore_axis_name="s")
    out = pl.kernel(
        _sc_body,
        out_type=jax.ShapeDtypeStruct((_N * _N * _R,), jnp.float32),
        mesh=mesh,
        scratch_types=[
            pltpu.VMEM((2 * _N * _R,), jnp.float32),
            pltpu.VMEM((2 * _N * _R,), jnp.float32),
            pltpu.SemaphoreType.DMA,
            pltpu.SemaphoreType.DMA,
        ],
    )(z2f)
    return out.reshape(_N, _N, _R)


# speedup vs baseline: 5.4276x; 5.4276x over previous
import jax
import jax.numpy as jnp
from jax import lax
from jax.experimental import pallas as pl
from jax.experimental.pallas import tpu as pltpu
from jax.experimental.pallas import tpu_sc as plsc

_N = 2048
_R = 8
_LANES = 16
_CHUNKS = _N * _R // _LANES
_NW = 32
_ROWS_PER_W = _N // _NW
_PAIRS = _ROWS_PER_W // 2
_BCHUNKS = _N // _LANES
_B = 8


def _sc_body(z1x_hbm, vcode_hbm, rowcode_hbm, rowpar_hbm, z2f_hbm, batch_hbm,
             out_hbm,
             z1_v, vcode_v, rowcode_v, rowpar_v, z2_v, batch_v,
             row_a, row_b, sem_a, sem_b):
    cid = lax.axis_index("c")
    sid = lax.axis_index("s")
    wid = sid * 2 + cid
    base = wid * _ROWS_PER_W

    pltpu.sync_copy(z2f_hbm, z2_v)
    pltpu.sync_copy(vcode_hbm, vcode_v)
    pltpu.sync_copy(batch_hbm, batch_v.at[pl.ds(0, _N)])
    pltpu.sync_copy(z1x_hbm.at[pl.ds(base, _ROWS_PER_W)], z1_v)
    pltpu.sync_copy(rowcode_hbm.at[pl.ds(base, _ROWS_PER_W)],
                    rowcode_v.at[pl.ds(0, _ROWS_PER_W)])
    pltpu.sync_copy(rowpar_hbm.at[pl.ds(base, _ROWS_PER_W)],
                    rowpar_v.at[pl.ds(0, _ROWS_PER_W)])

    iota = lax.iota(jnp.int32, _LANES)
    basev = jnp.where(iota % _R == 0, 1.0, 0.0).astype(jnp.float32)

    def _count_lt(v):
        def sb(_, lohi):
            lo, hi = lohi
            mid = (lo + hi) // 2
            bm = batch_v[pl.ds(mid, _LANES)][0]
            go_right = bm < v
            lo = jnp.where(go_right, mid + 1, lo)
            hi = jnp.where(go_right, hi, mid)
            return lo, hi
        lo, _hi = lax.fori_loop(0, 11, sb, (jnp.int32(0), jnp.int32(_N)))
        return lo

    starts = [jnp.int32(0)]
    for v in range(1, _B):
        starts.append(_count_lt(v))
    starts.append(jnp.int32(_N))

    def init_body(j, c):
        row_a[pl.ds(j * _LANES, _LANES)] = basev
        row_b[pl.ds(j * _LANES, _LANES)] = basev
        return c
    lax.fori_loop(0, _CHUNKS, init_body, 0)

    def process_row(i, rowbuf, d0, d1):
        r = base + i
        c_rf = rowcode_v[pl.ds(i, _LANES)][0]
        c_rv = jnp.broadcast_to(c_rf, (_LANES,))
        z1v = z1_v[i, :]
        bri = batch_v[pl.ds(r, _LANES)][0]
        row_ok = c_rf >= 0.0

        bs = jnp.int32(0)
        be = jnp.int32(0)
        for v in range(_B):
            m = jnp.logical_and(row_ok, bri == v)
            bs = jnp.where(m, starts[v], bs)
            be = jnp.where(m, starts[v + 1], be)
        c0 = bs // 2
        c1 = (be + 1) // 2

        def rest(j, c):
            rowbuf[pl.ds(j * _LANES, _LANES)] = basev
            return c
        lax.fori_loop(d0, jnp.minimum(d1, c0), rest, 0)
        lax.fori_loop(jnp.maximum(d0, c1), d1, rest, 0)

        def cb(j, c):
            off = j * _LANES
            z2c = z2_v[pl.ds(off, _LANES)]
            vcc = vcode_v[pl.ds(off, _LANES)]
            rowbuf[pl.ds(off, _LANES)] = jnp.where(vcc == c_rv, z1v * z2c,
                                                   basev)
            return c
        lax.fori_loop(c0, c1, cb, 0)

        jd = r // 2
        parv = rowpar_v[pl.ds(i, _LANES)][0]
        rmv = jnp.broadcast_to(parv, (_LANES,))
        iota_l = lax.iota(jnp.int32, _LANES)
        halff = jnp.where(iota_l < _R, 0.0, 1.0).astype(jnp.float32)
        cur = rowbuf[pl.ds(jd * _LANES, _LANES)]
        rowbuf[pl.ds(jd * _LANES, _LANES)] = jnp.where(halff == rmv, basev,
                                                       cur)
        return c0, c1

    d0a, d1a = process_row(0, row_a, jnp.int32(0), jnp.int32(0))
    pltpu.make_async_copy(row_a, out_hbm.at[base], sem_a).start()
    d0b, d1b = process_row(1, row_b, jnp.int32(0), jnp.int32(0))
    pltpu.make_async_copy(row_b, out_hbm.at[base + 1], sem_b).start()

    def pair_body(g, carry):
        d0a, d1a, d0b, d1b = carry
        pltpu.make_async_copy(row_a, out_hbm.at[base], sem_a).wait()
        d0a, d1a = process_row(2 * g, row_a, d0a, d1a)
        pltpu.make_async_copy(row_a, out_hbm.at[base + 2 * g], sem_a).start()
        pltpu.make_async_copy(row_b, out_hbm.at[base], sem_b).wait()
        d0b, d1b = process_row(2 * g + 1, row_b, d0b, d1b)
        pltpu.make_async_copy(row_b, out_hbm.at[base + 2 * g + 1],
                              sem_b).start()
        return d0a, d1a, d0b, d1b

    lax.fori_loop(1, _PAIRS, pair_body, (d0a, d1a, d0b, d1b))

    pltpu.make_async_copy(row_a, out_hbm.at[base], sem_a).wait()
    pltpu.make_async_copy(row_b, out_hbm.at[base], sem_b).wait()


def kernel(z1, z2, seg_matrix, cls_label, batch):
    del seg_matrix
    node_mask = (cls_label != 24) & (cls_label != 25) & (cls_label != 26)
    bf = batch.astype(jnp.float32)
    vcode = jnp.repeat(jnp.where(node_mask, bf, -1.0), _R)
    rowcode = jnp.where(node_mask, bf, -2.0)
    rowpar = (jnp.arange(_N) % 2).astype(jnp.float32)
    z1x = jnp.concatenate([z1, z1], axis=1)
    z2f = z2.reshape(-1)
    batch_i = batch.astype(jnp.int32)

    mesh = plsc.VectorSubcoreMesh(core_axis_name="c", subcore_axis_name="s")
    out = pl.kernel(
        _sc_body,
        out_type=jax.ShapeDtypeStruct((_N, _N * _R), jnp.float32),
        mesh=mesh,
        scratch_types=[
            pltpu.VMEM((_ROWS_PER_W, _LANES), jnp.float32),
            pltpu.VMEM((_N * _R,), jnp.float32),
            pltpu.VMEM((_ROWS_PER_W + _LANES,), jnp.float32),
            pltpu.VMEM((_ROWS_PER_W + _LANES,), jnp.float32),
            pltpu.VMEM((_N * _R,), jnp.float32),
            pltpu.VMEM((_N + _LANES,), jnp.int32),
            pltpu.VMEM((_N * _R,), jnp.float32),
            pltpu.VMEM((_N * _R,), jnp.float32),
            pltpu.SemaphoreType.DMA,
            pltpu.SemaphoreType.DMA,
        ],
    )(z1x, vcode, rowcode, rowpar, z2f, batch_i)
    return out.reshape(_N, _N, _R)
